# baseline (speedup 1.0000x reference)
import functools

import jax
import jax.numpy as jnp
from jax import lax
from jax.experimental import pallas as pl
from jax.experimental.pallas import tpu as pltpu
from jax.experimental.pallas import tpu_sc as plsc

_SCHED = ((0, 32, 0), (32, 24, 2), (56, 32, 0), (88, 24, 2), (112, 16, 0))
_TS_ROWS = (32,)
_SP_ROWS = (24,)


@functools.lru_cache(maxsize=None)
def _make_copy(seq: int, d: int, dtype_name: str):
    dtype = jnp.dtype(dtype_name)
    info = plsc.get_sparse_core_info()
    nc, ns = info.num_cores, info.num_subcores
    nw = nc * ns
    rows_per_w = seq // nw
    assert seq == nw * rows_per_w
    assert rows_per_w == sum(n for _, n, _ in _SCHED)
    nchunks = len(_SCHED)
    sp_rows = sum(_SP_ROWS)

    mesh = plsc.VectorSubcoreMesh(core_axis_name="c", subcore_axis_name="s")

    @functools.partial(
        pl.kernel,
        mesh=mesh,
        out_type=jax.ShapeDtypeStruct((seq, d), dtype),
        scratch_types=[
            pltpu.VMEM((_TS_ROWS[0], d), dtype),
            pltpu.MemorySpace.VMEM_SHARED((ns, sp_rows, d), dtype),
            pltpu.SemaphoreType.DMA((4,)),
            pltpu.SemaphoreType.DMA((4,)),
        ],
    )
    def copy_kernel(table_hbm, out_hbm, ts_a, spmem, in_sems, out_sems):
        cid = lax.axis_index("c")
        sid = lax.axis_index("s")
        wid = sid * nc + cid
        base = wid * rows_per_w

        def buf(k, n):
            if k == 0:
                return ts_a.at[pl.ds(0, n)]
            return spmem.at[sid, pl.ds(0, n)]

        def issue_in(c):
            off, n, k = _SCHED[c]
            return pltpu.async_copy(
                table_hbm.at[pl.ds(base + off, n)], buf(k, n), in_sems.at[k]
            )

        def issue_out(c):
            off, n, k = _SCHED[c]
            return pltpu.async_copy(
                buf(k, n), out_hbm.at[pl.ds(base + off, n)], out_sems.at[k]
            )

        in_cp = [None] * nchunks
        out_cp = [None] * nchunks
        last_out_for_buf = {}
        in_cp[0] = issue_in(0)
        in_cp[1] = issue_in(1)
        for c in range(nchunks):
            in_cp[c].wait()
            out_cp[c] = issue_out(c)
            last_out_for_buf[_SCHED[c][2]] = out_cp[c]
            nxt = c + 2
            if nxt < nchunks:
                k_nxt = _SCHED[nxt][2]
                if k_nxt in last_out_for_buf:
                    last_out_for_buf[k_nxt].wait()
                in_cp[nxt] = issue_in(nxt)
        for cp in last_out_for_buf.values():
            cp.wait()

    return copy_kernel


def kernel(x, emb_weight):
    seq = x.shape[1]
    return _make_copy(seq, emb_weight.shape[1], emb_weight.dtype.name)(emb_weight)
